# baseline (speedup 1.0000x reference)
import functools

import jax
import jax.numpy as jnp
from jax import lax
from jax.experimental import pallas as pl
from jax.experimental.pallas import tpu as pltpu
from jax.experimental.pallas import tpu_sc as plsc

_CHUNK = 128
_NBUF = 5


def _emb_lookup(ids3, table, n_steps, nc, nw):
    D = table.shape[1]
    N = nw * n_steps * _CHUNK
    mesh = plsc.VectorSubcoreMesh(core_axis_name="c", subcore_axis_name="s")

    @functools.partial(
        pl.kernel,
        mesh=mesh,
        out_type=jax.ShapeDtypeStruct((N, D), jnp.float32),
        compiler_params=pltpu.CompilerParams(use_tc_tiling_on_sc=False),
        scratch_types=[
            pltpu.VMEM((n_steps, _CHUNK), jnp.int32),
            pltpu.VMEM((_NBUF, _CHUNK, D), jnp.float32),
            pltpu.SemaphoreType.DMA,
            pltpu.SemaphoreType.DMA,
        ],
    )
    def emb(ids_hbm, table_hbm, out_hbm, idx_v, rows_v, gsem, osem):
        wid = lax.axis_index("s") * nc + lax.axis_index("c")
        base = wid * (n_steps * _CHUNK)
        pltpu.sync_copy(ids_hbm.at[wid], idx_v)

        def gather_copy(ci, buf):
            return pltpu.make_async_copy(
                table_hbm.at[idx_v.at[ci]], rows_v.at[buf], gsem
            )

        def out_copy(ci, buf):
            return pltpu.make_async_copy(
                rows_v.at[buf],
                out_hbm.at[pl.ds(base + ci * _CHUNK, _CHUNK)],
                osem,
            )

        for b in range(_NBUF):
            gather_copy(b, b).start()

        def body(g):
            for i in range(_NBUF):
                j = g + i
                gather_copy(j, i).wait()
                out_copy(j, i).start()
                out_copy(j, i).wait()
                gather_copy(j + _NBUF, i).start()

        pl.loop(0, n_steps - _NBUF, step=_NBUF)(body)

        for i in range(_NBUF):
            j = n_steps - _NBUF + i
            gather_copy(j, i).wait()
            out_copy(j, i).start()
            out_copy(j, i).wait()

    return emb(ids3, table)


def kernel(ids, table):
    B, H = ids.shape
    V, D = table.shape
    N = B * H
    info = plsc.get_sparse_core_info()
    nc, ns = info.num_cores, info.num_subcores
    nw = nc * ns
    n_per_w = N // nw
    n_steps = n_per_w // _CHUNK
    ids3 = ids.reshape(nw, n_steps, _CHUNK).astype(jnp.int32)
    out = _emb_lookup(ids3, table, n_steps, nc, nw)
    return out.reshape(B, H, D)
